# Initial kernel scaffold; baseline (speedup 1.0000x reference)
import jax
import jax.numpy as jnp
from jax.experimental import pallas as pl


def kernel(x, emb, bw0, bb0, bw1, bb1, bw2, bb2, tw0, tb0, tw1, tb1, tw2, tb2, tw3, tb3, tw4, tb4):
    raise NotImplementedError("write your pallas kernel here")


# speedup vs baseline: 10.3279x; 10.3279x over previous
import functools

import jax
import jax.numpy as jnp
import numpy as np
from jax import lax
from jax.experimental import pallas as pl
from jax.experimental.pallas import tpu as pltpu
from jax.experimental.pallas import tpu_sc as plsc

_VOCAB = 1000000
_EMBED = 128
_B = 16384
_ND = 13
_NS = 26

_NC = 2
_NSUB = 16
_NW = _NC * _NSUB
_CHUNK = 128
_ROWS = _B * _NS
_CH = _ROWS // (_NW * _CHUNK)


def _sc_gather(emb, idx3):
    mesh = plsc.VectorSubcoreMesh(core_axis_name="c", subcore_axis_name="s")

    @functools.partial(
        pl.kernel,
        out_type=jax.ShapeDtypeStruct((_ROWS // _CHUNK, _CHUNK, _EMBED), jnp.float32),
        mesh=mesh,
        scratch_types=[
            pltpu.VMEM((_CH, _CHUNK), jnp.int32),
            pltpu.VMEM((_CHUNK, _EMBED), jnp.float32),
            pltpu.VMEM((_CHUNK, _EMBED), jnp.float32),
            pltpu.SemaphoreType.DMA,
            pltpu.SemaphoreType.DMA,
        ],
    )
    def body(emb_hbm, idx_hbm, out_hbm, idx_v, buf0, buf1, sem0, sem1):
        wid = lax.axis_index("s") * _NC + lax.axis_index("c")
        pltpu.sync_copy(idx_hbm.at[wid], idx_v)
        cbase = wid * _CH
        pltpu.make_async_copy(emb_hbm.at[idx_v.at[0]], buf0, sem0).start()

        def step(t, carry):
            j0 = t * 2
            pltpu.make_async_copy(emb_hbm.at[idx_v.at[j0 + 1]], buf1, sem1).start()
            pltpu.make_async_copy(emb_hbm.at[idx_v.at[j0]], buf0, sem0).wait()
            pltpu.sync_copy(buf0, out_hbm.at[cbase + j0])

            @pl.when(j0 + 2 < _CH)
            def _():
                pltpu.make_async_copy(emb_hbm.at[idx_v.at[j0 + 2]], buf0, sem0).start()

            pltpu.make_async_copy(emb_hbm.at[idx_v.at[j0 + 1]], buf1, sem1).wait()
            pltpu.sync_copy(buf1, out_hbm.at[cbase + j0 + 1])
            return carry

        lax.fori_loop(0, _CH // 2, step, 0)

    return body(emb, idx3)


_BB = 256


def _tc_body(dense, gath, w0, b0, w1, b1, w2, b2,
             wh, w729, tb0, wt1, tb1, wt2, tb2, wt3, tb3, wt4, tb4, out):
    f32 = jnp.float32
    h = jnp.maximum(jnp.dot(dense[...], w0[...], preferred_element_type=f32) + b0[...], 0.0)
    h = jnp.maximum(jnp.dot(h, w1[...], preferred_element_type=f32) + b1[...], 0.0)
    h2 = jnp.maximum(jnp.dot(h, w2[...], preferred_element_type=f32) + b2[...], 0.0)
    e = gath[...].reshape(_BB, _NS, _EMBED)
    comb = jnp.concatenate([h2.reshape(_BB, 1, _EMBED), e], axis=1)
    inter = lax.dot_general(comb, comb, (((2,), (2,)), ((0,), (0,))),
                            preferred_element_type=f32)
    interf = inter.reshape(_BB, 729)
    t = jnp.dot(h2, wh[...], preferred_element_type=f32)
    t = t + jnp.dot(interf, w729[...], preferred_element_type=f32)
    t = jnp.maximum(t + tb0[...], 0.0)
    t = jnp.maximum(jnp.dot(t, wt1[...], preferred_element_type=f32) + tb1[...], 0.0)
    t = jnp.maximum(jnp.dot(t, wt2[...], preferred_element_type=f32) + tb2[...], 0.0)
    t = jnp.maximum(jnp.dot(t, wt3[...], preferred_element_type=f32) + tb3[...], 0.0)
    out[...] = jnp.dot(t, wt4[...], preferred_element_type=f32) + tb4[...]


def _tc_fused(dense, gath, weights):
    grid = (_B // _BB,)

    def blk(shape):
        return pl.BlockSpec(shape, lambda i: (i, 0))

    def rep(shape):
        return pl.BlockSpec(shape, lambda i: (0, 0))

    in_specs = [blk((_BB, _ND)), blk((_BB, _NS * _EMBED))]
    for w in weights:
        in_specs.append(rep(w.shape))
    return pl.pallas_call(
        _tc_body,
        grid=grid,
        in_specs=in_specs,
        out_specs=blk((_BB, 1)),
        out_shape=jax.ShapeDtypeStruct((_B, 1), jnp.float32),
    )(dense, gath, *weights)


def kernel(x, emb, bw0, bb0, bw1, bb1, bw2, bb2,
           tw0, tb0, tw1, tb1, tw2, tb2, tw3, tb3, tw4, tb4):
    dense = x[:, :_ND]
    idx = (x[:, _ND:].astype(jnp.int32) % _VOCAB).reshape(_NW, _CH, _CHUNK)
    gathered = _sc_gather(emb, idx).reshape(_B, _NS * _EMBED)

    iu, ku = np.triu_indices(_NS + 1)
    rows = jnp.asarray(iu * (_NS + 1) + ku, dtype=jnp.int32)
    w729 = jnp.zeros((729, 1024), jnp.float32).at[rows].set(tw0[:, _EMBED:].T)

    weights = (
        bw0.T, bb0.reshape(1, -1), bw1.T, bb1.reshape(1, -1), bw2.T, bb2.reshape(1, -1),
        tw0[:, :_EMBED].T, w729, tb0.reshape(1, -1),
        tw1.T, tb1.reshape(1, -1), tw2.T, tb2.reshape(1, -1),
        tw3.T, tb3.reshape(1, -1), tw4.T, tb4.reshape(1, -1),
    )
    return _tc_fused(dense, gathered, weights)


# speedup vs baseline: 13.9474x; 1.3505x over previous
import functools

import jax
import jax.numpy as jnp
import numpy as np
from jax import lax
from jax.experimental import pallas as pl
from jax.experimental.pallas import tpu as pltpu
from jax.experimental.pallas import tpu_sc as plsc

_VOCAB = 1000000
_EMBED = 128
_B = 16384
_ND = 13
_NS = 26

_NC = 2
_NSUB = 16
_NW = _NC * _NSUB
_CHUNK = 128
_ROWS = _B * _NS
_CH = _ROWS // (_NW * _CHUNK)


def _sc_gather(emb, idx3):
    mesh = plsc.VectorSubcoreMesh(core_axis_name="c", subcore_axis_name="s")

    @functools.partial(
        pl.kernel,
        out_type=jax.ShapeDtypeStruct((_ROWS // _CHUNK, _CHUNK, _EMBED), jnp.float32),
        mesh=mesh,
        scratch_types=[
            pltpu.VMEM((_CH, _CHUNK), jnp.int32),
            pltpu.VMEM((_CHUNK, _EMBED), jnp.float32),
            pltpu.VMEM((_CHUNK, _EMBED), jnp.float32),
            pltpu.VMEM((_CHUNK, _EMBED), jnp.float32),
            pltpu.VMEM((_CHUNK, _EMBED), jnp.float32),
            pltpu.SemaphoreType.DMA,
            pltpu.SemaphoreType.DMA,
            pltpu.SemaphoreType.DMA,
            pltpu.SemaphoreType.DMA,
        ],
    )
    def body(emb_hbm, idx_hbm, out_hbm, idx_v,
             buf0, buf1, buf2, buf3, sem0, sem1, sem2, sem3):
        bufs = (buf0, buf1, buf2, buf3)
        sems = (sem0, sem1, sem2, sem3)
        wid = lax.axis_index("s") * _NC + lax.axis_index("c")
        pltpu.sync_copy(idx_hbm.at[wid], idx_v)
        cbase = wid * _CH
        for u in range(3):
            pltpu.make_async_copy(emb_hbm.at[idx_v.at[u]], bufs[u], sems[u]).start()

        def step(t, carry):
            j0 = t * 4
            for u in range(4):
                j = j0 + u
                nb = bufs[(u + 3) % 4]
                ns = sems[(u + 3) % 4]

                @pl.when(j + 3 < _CH)
                def _():
                    pltpu.make_async_copy(emb_hbm.at[idx_v.at[j + 3]], nb, ns).start()

                pltpu.make_async_copy(emb_hbm.at[idx_v.at[j]], bufs[u], sems[u]).wait()
                pltpu.sync_copy(bufs[u], out_hbm.at[cbase + j])
            return carry

        lax.fori_loop(0, _CH // 4, step, 0)

    return body(emb, idx3)


_BB = 256


def _tc_body(dense, gath, w0, b0, w1, b1, w2, b2,
             wh, w729, tb0, wt1, tb1, wt2, tb2, wt3, tb3, wt4, tb4, out):
    f32 = jnp.float32
    h = jnp.maximum(jnp.dot(dense[...], w0[...], preferred_element_type=f32) + b0[...], 0.0)
    h = jnp.maximum(jnp.dot(h, w1[...], preferred_element_type=f32) + b1[...], 0.0)
    h2 = jnp.maximum(jnp.dot(h, w2[...], preferred_element_type=f32) + b2[...], 0.0)
    e = gath[...].reshape(_BB * _NS, _EMBED).reshape(_BB, _NS, _EMBED)
    comb = jnp.concatenate([h2.reshape(_BB, 1, _EMBED), e], axis=1)
    inter = lax.dot_general(comb, comb, (((2,), (2,)), ((0,), (0,))),
                            preferred_element_type=f32)
    interf = inter.reshape(_BB, 729)
    t = jnp.dot(h2, wh[...], preferred_element_type=f32)
    t = t + jnp.dot(interf, w729[...], preferred_element_type=f32)
    t = jnp.maximum(t + tb0[...], 0.0)
    t = jnp.maximum(jnp.dot(t, wt1[...], preferred_element_type=f32) + tb1[...], 0.0)
    t = jnp.maximum(jnp.dot(t, wt2[...], preferred_element_type=f32) + tb2[...], 0.0)
    t = jnp.maximum(jnp.dot(t, wt3[...], preferred_element_type=f32) + tb3[...], 0.0)
    out[...] = jnp.dot(t, wt4[...], preferred_element_type=f32) + tb4[...]


def _tc_fused(dense, gath3, weights):
    grid = (_B // _BB,)
    nch = _BB * _NS // _CHUNK

    def blk(shape):
        return pl.BlockSpec(shape, lambda i: (i, 0))

    def rep(shape):
        return pl.BlockSpec(shape, lambda i: (0,) * len(shape))

    in_specs = [blk((_BB, _ND)),
                pl.BlockSpec((nch, _CHUNK, _EMBED), lambda i: (i, 0, 0))]
    for w in weights:
        in_specs.append(rep(w.shape))
    return pl.pallas_call(
        _tc_body,
        grid=grid,
        in_specs=in_specs,
        out_specs=blk((_BB, 1)),
        out_shape=jax.ShapeDtypeStruct((_B, 1), jnp.float32),
    )(dense, gath3, *weights)


def kernel(x, emb, bw0, bb0, bw1, bb1, bw2, bb2,
           tw0, tb0, tw1, tb1, tw2, tb2, tw3, tb3, tw4, tb4):
    dense = x[:, :_ND]
    idx = (x[:, _ND:].astype(jnp.int32) % _VOCAB).reshape(_NW, _CH, _CHUNK)
    gathered = _sc_gather(emb, idx)

    iu, ku = np.triu_indices(_NS + 1)
    rows = jnp.asarray(iu * (_NS + 1) + ku, dtype=jnp.int32)
    w729 = jnp.zeros((729, 1024), jnp.float32).at[rows].set(tw0[:, _EMBED:].T)

    weights = (
        bw0.T, bb0.reshape(1, -1), bw1.T, bb1.reshape(1, -1), bw2.T, bb2.reshape(1, -1),
        tw0[:, :_EMBED].T, w729, tb0.reshape(1, -1),
        tw1.T, tb1.reshape(1, -1), tw2.T, tb2.reshape(1, -1),
        tw3.T, tb3.reshape(1, -1), tw4.T, tb4.reshape(1, -1),
    )
    return _tc_fused(dense, gathered, weights)


# speedup vs baseline: 15.6454x; 1.1217x over previous
import functools

import jax
import jax.numpy as jnp
import numpy as np
from jax import lax
from jax.experimental import pallas as pl
from jax.experimental.pallas import tpu as pltpu
from jax.experimental.pallas import tpu_sc as plsc

_VOCAB = 1000000
_EMBED = 128
_B = 16384
_ND = 13
_NS = 26

_NC = 2
_NSUB = 16
_NW = _NC * _NSUB

_NSLICE = 4
_BS = _B // _NSLICE
_ROWS_S = _BS * _NS
_CHUNK = 104
_CH_S = _ROWS_S // (_NW * _CHUNK)


def _sc_gather(emb, idx3):
    mesh = plsc.VectorSubcoreMesh(core_axis_name="c", subcore_axis_name="s")

    @functools.partial(
        pl.kernel,
        out_type=jax.ShapeDtypeStruct((_ROWS_S // _CHUNK, _CHUNK, _EMBED), jnp.float32),
        mesh=mesh,
        scratch_types=[
            pltpu.VMEM((_CH_S, _CHUNK), jnp.int32),
            pltpu.VMEM((_CHUNK, _EMBED), jnp.float32),
            pltpu.VMEM((_CHUNK, _EMBED), jnp.float32),
            pltpu.VMEM((_CHUNK, _EMBED), jnp.float32),
            pltpu.VMEM((_CHUNK, _EMBED), jnp.float32),
            pltpu.SemaphoreType.DMA,
            pltpu.SemaphoreType.DMA,
            pltpu.SemaphoreType.DMA,
            pltpu.SemaphoreType.DMA,
        ],
    )
    def body(emb_hbm, idx_hbm, out_hbm, idx_v,
             buf0, buf1, buf2, buf3, sem0, sem1, sem2, sem3):
        bufs = (buf0, buf1, buf2, buf3)
        sems = (sem0, sem1, sem2, sem3)
        wid = lax.axis_index("s") * _NC + lax.axis_index("c")
        pltpu.sync_copy(idx_hbm.at[wid], idx_v)
        cbase = wid * _CH_S
        for u in range(3):
            pltpu.make_async_copy(emb_hbm.at[idx_v.at[u]], bufs[u], sems[u]).start()

        def step(t, carry):
            j0 = t * 4
            for u in range(4):
                j = j0 + u
                nb = bufs[(u + 3) % 4]
                ns = sems[(u + 3) % 4]

                @pl.when(j + 3 < _CH_S)
                def _():
                    pltpu.make_async_copy(emb_hbm.at[idx_v.at[j + 3]], nb, ns).start()

                pltpu.make_async_copy(emb_hbm.at[idx_v.at[j]], bufs[u], sems[u]).wait()
                pltpu.sync_copy(bufs[u], out_hbm.at[cbase + j])
            return carry

        lax.fori_loop(0, _CH_S // 4, step, 0)

    return body(emb, idx3)


_BB = 256


def _tc_body(dense, gath, w0, b0, w1, b1, w2, b2,
             wh, w729, tb0, wt1, tb1, wt2, tb2, wt3, tb3, wt4, tb4, out):
    f32 = jnp.float32
    h = jnp.maximum(jnp.dot(dense[...], w0[...], preferred_element_type=f32) + b0[...], 0.0)
    h = jnp.maximum(jnp.dot(h, w1[...], preferred_element_type=f32) + b1[...], 0.0)
    h2 = jnp.maximum(jnp.dot(h, w2[...], preferred_element_type=f32) + b2[...], 0.0)
    e = gath[...].reshape(_BB * _NS, _EMBED).reshape(_BB, _NS, _EMBED)
    comb = jnp.concatenate([h2.reshape(_BB, 1, _EMBED), e], axis=1)
    inter = lax.dot_general(comb, comb, (((2,), (2,)), ((0,), (0,))),
                            preferred_element_type=f32)
    interf = inter.reshape(_BB, 729)
    t = jnp.dot(h2, wh[...], preferred_element_type=f32)
    t = t + jnp.dot(interf, w729[...], preferred_element_type=f32)
    t = jnp.maximum(t + tb0[...], 0.0)
    t = jnp.maximum(jnp.dot(t, wt1[...], preferred_element_type=f32) + tb1[...], 0.0)
    t = jnp.maximum(jnp.dot(t, wt2[...], preferred_element_type=f32) + tb2[...], 0.0)
    t = jnp.maximum(jnp.dot(t, wt3[...], preferred_element_type=f32) + tb3[...], 0.0)
    out[...] = jnp.dot(t, wt4[...], preferred_element_type=f32) + tb4[...]


def _tc_fused(dense, gath3, weights):
    grid = (_BS // _BB,)
    nch = _BB * _NS // _CHUNK

    def blk(shape):
        return pl.BlockSpec(shape, lambda i: (i, 0))

    def rep(shape):
        return pl.BlockSpec(shape, lambda i: (0,) * len(shape))

    in_specs = [blk((_BB, _ND)),
                pl.BlockSpec((nch, _CHUNK, _EMBED), lambda i: (i, 0, 0))]
    for w in weights:
        in_specs.append(rep(w.shape))
    return pl.pallas_call(
        _tc_body,
        grid=grid,
        in_specs=in_specs,
        out_specs=blk((_BB, 1)),
        out_shape=jax.ShapeDtypeStruct((_BS, 1), jnp.float32),
    )(dense, gath3, *weights)


def kernel(x, emb, bw0, bb0, bw1, bb1, bw2, bb2,
           tw0, tb0, tw1, tb1, tw2, tb2, tw3, tb3, tw4, tb4):
    dense = x[:, :_ND]
    idx_all = x[:, _ND:].astype(jnp.int32) % _VOCAB

    iu, ku = np.triu_indices(_NS + 1)
    rows = jnp.asarray(iu * (_NS + 1) + ku, dtype=jnp.int32)
    w729 = jnp.zeros((729, 1024), jnp.float32).at[rows].set(tw0[:, _EMBED:].T)

    weights = (
        bw0.T, bb0.reshape(1, -1), bw1.T, bb1.reshape(1, -1), bw2.T, bb2.reshape(1, -1),
        tw0[:, :_EMBED].T, w729, tb0.reshape(1, -1),
        tw1.T, tb1.reshape(1, -1), tw2.T, tb2.reshape(1, -1),
        tw3.T, tb3.reshape(1, -1), tw4.T, tb4.reshape(1, -1),
    )

    parts = []
    for s in range(_NSLICE):
        idx_s = idx_all[s * _BS:(s + 1) * _BS].reshape(_NW, _CH_S, _CHUNK)
        g = _sc_gather(emb, idx_s)
        parts.append(_tc_fused(dense[s * _BS:(s + 1) * _BS], g, weights))
    return jnp.concatenate(parts, axis=0)
